# baseline (speedup 1.0000x reference)
import functools

import jax
import jax.numpy as jnp
from jax import lax
from jax.experimental import pallas as pl
from jax.experimental.pallas import tpu as pltpu
from jax.experimental.pallas import tpu_sc as plsc

NC = 2
NS = 16
NW = NC * NS
LANES = 16


def kernel(user, item, user_table, item_table):
    B = user.shape[0]
    D = user_table.shape[1]
    BPW = B // NW
    CH = 128
    NCH = BPW // CH

    user_idx = user.astype(jnp.int32).reshape(NW, NCH, CH)
    item_idx = item.astype(jnp.int32).reshape(NW, NCH, CH)

    mesh = plsc.VectorSubcoreMesh(core_axis_name="c", subcore_axis_name="s")

    @functools.partial(
        pl.kernel,
        out_type=jax.ShapeDtypeStruct((B,), jnp.float32),
        mesh=mesh,
        scratch_types=[
            pltpu.VMEM((NCH, CH), jnp.int32),
            pltpu.VMEM((NCH, CH), jnp.int32),
            pltpu.VMEM((BPW, D), jnp.float32),
            pltpu.VMEM((BPW, D), jnp.float32),
            pltpu.VMEM((BPW,), jnp.float32),
            pltpu.SemaphoreType.DMA,
        ],
        compiler_params=pltpu.CompilerParams(
            needs_layout_passes=False, use_tc_tiling_on_sc=False),
    )
    def _emb_dot(uidx_hbm, iidx_hbm, utab_hbm, itab_hbm, out_hbm,
                 uidx_v, iidx_v, urows_v, irows_v, out_v, sem):
        wid = lax.axis_index("s") * NC + lax.axis_index("c")
        base = wid * BPW

        pltpu.sync_copy(uidx_hbm.at[wid], uidx_v)
        pltpu.sync_copy(iidx_hbm.at[wid], iidx_v)

        copies = []
        for j in range(NCH):
            copies.append(pltpu.async_copy(
                utab_hbm.at[uidx_v.at[j]],
                urows_v.at[pl.ds(j * CH, CH)], sem))
            copies.append(pltpu.async_copy(
                itab_hbm.at[iidx_v.at[j]],
                irows_v.at[pl.ds(j * CH, CH)], sem))
        for c in copies:
            c.wait()

        iota = lax.iota(jnp.int32, LANES)

        def body(r, carry):
            base_row = r * LANES
            res = jnp.zeros((LANES,), jnp.float32)
            for k in range(LANES):
                b = base_row + k
                acc = None
                for g in range(D // LANES):
                    ug = urows_v[b, pl.ds(g * LANES, LANES)]
                    ig = irows_v[b, pl.ds(g * LANES, LANES)]
                    prod = ug * ig
                    acc = prod if acc is None else acc + prod
                s = jnp.broadcast_to(jnp.sum(acc), (LANES,))
                res = jnp.where(iota == k, s, res)
            out_v[pl.ds(base_row, LANES)] = res
            return carry

        lax.fori_loop(0, BPW // LANES, body, 0)

        pltpu.sync_copy(out_v, out_hbm.at[pl.ds(base, BPW)])

    return _emb_dot(user_idx, item_idx, user_table, item_table)


# speedup vs baseline: 2.7703x; 2.7703x over previous
import functools

import jax
import jax.numpy as jnp
from jax import lax
from jax.experimental import pallas as pl
from jax.experimental.pallas import tpu as pltpu
from jax.experimental.pallas import tpu_sc as plsc

NC = 2
NS = 16
NW = NC * NS
LANES = 16
Q = 4


def kernel(user, item, user_table, item_table):
    B = user.shape[0]
    D = user_table.shape[1]
    BPW = B // NW

    ut = user_table.T
    it = item_table.T

    mesh = plsc.VectorSubcoreMesh(core_axis_name="c", subcore_axis_name="s")

    @functools.partial(
        pl.kernel,
        out_type=jax.ShapeDtypeStruct((B,), jnp.float32),
        mesh=mesh,
        scratch_types=[
            pltpu.VMEM((BPW,), jnp.int32),
            pltpu.VMEM((BPW,), jnp.int32),
            pltpu.VMEM((Q, D, 128), jnp.float32),
            pltpu.VMEM((Q, D, 128), jnp.float32),
            pltpu.VMEM((BPW,), jnp.float32),
            pltpu.SemaphoreType.DMA((Q,)),
            pltpu.SemaphoreType.DMA((Q,)),
        ],
        compiler_params=pltpu.CompilerParams(
            needs_layout_passes=False, use_tc_tiling_on_sc=True),
    )
    def _emb_dot(uidx_hbm, iidx_hbm, ut_hbm, it_hbm, out_hbm,
                 uidx_v, iidx_v, u_pan, i_pan, out_v, sem_u, sem_i):
        wid = lax.axis_index("s") * NC + lax.axis_index("c")
        base = wid * BPW

        pltpu.sync_copy(uidx_hbm.at[pl.ds(base, BPW)], uidx_v)
        pltpu.sync_copy(iidx_hbm.at[pl.ds(base, BPW)], iidx_v)

        iota = lax.iota(jnp.int32, LANES)
        zero = jnp.zeros((LANES,), jnp.int32)

        def extract(vref, k):
            vec = vref[pl.ds((k // LANES) * LANES, LANES)]
            return jnp.sum(jnp.where(iota == (k % LANES), vec, zero))

        def issue(k, q):
            cu = (extract(uidx_v, k) // 128) * 128
            ci = (extract(iidx_v, k) // 128) * 128
            pltpu.async_copy(ut_hbm.at[:, pl.ds(cu, 128)], u_pan.at[q],
                             sem_u.at[q])
            pltpu.async_copy(it_hbm.at[:, pl.ds(ci, 128)], i_pan.at[q],
                             sem_i.at[q])

        for q in range(Q):
            issue(q, q)

        def body(k0, carry):
            for q in range(Q):
                k = k0 * Q + q
                pltpu.make_async_copy(ut_hbm.at[:, pl.ds(0, 128)],
                                      u_pan.at[q], sem_u.at[q]).wait()
                pltpu.make_async_copy(it_hbm.at[:, pl.ds(0, 128)],
                                      i_pan.at[q], sem_i.at[q]).wait()
                ju = jnp.broadcast_to(extract(uidx_v, k) % 128, (LANES,))
                ji = jnp.broadcast_to(extract(iidx_v, k) % 128, (LANES,))
                acc = None
                for g in range(D // LANES):
                    dvec = g * LANES + iota
                    ug = plsc.load_gather(u_pan.at[q], [dvec, ju])
                    ig = plsc.load_gather(i_pan.at[q], [dvec, ji])
                    prod = ug * ig
                    acc = prod if acc is None else acc + prod
                s = jnp.broadcast_to(jnp.sum(acc), (LANES,))
                kn = k + Q
                pl.when(kn < BPW)(lambda: issue(kn, q))
                blk = (k // LANES) * LANES
                cur = out_v[pl.ds(blk, LANES)]
                out_v[pl.ds(blk, LANES)] = jnp.where(iota == (k % LANES),
                                                     s, cur)
            return carry

        lax.fori_loop(0, BPW // Q, body, 0)

        pltpu.sync_copy(out_v, out_hbm.at[pl.ds(base, BPW)])

    return _emb_dot(user.astype(jnp.int32), item.astype(jnp.int32), ut, it)
